# baseline (speedup 1.0000x reference)
import functools

import jax
from jax import lax
import jax.numpy as jnp
from jax.experimental import pallas as pl
from jax.experimental.pallas import tpu as pltpu

_B, _N, _C = 8, 20000, 80
_NMS_T = 0.5
_SCORE_T = 0.05
_MAXDET = 100
_R, _L = 160, 128
_NPAD = _R * _L
_NEG_INF = float("-inf")
_NG = 4
_NH = _N // 2
_GS = _C // _NG

_INTERPRET = False


def _scores_body(cls_ref, s_ref, l_ref, b, h):
    x = cls_ref[0]
    xt = x.T
    m = jnp.max(xt, axis=0)

    ci = lax.broadcasted_iota(jnp.int32, (_NG, _C), 1)
    ki = lax.broadcasted_iota(jnp.int32, (_NG, _C), 0)
    j = ci - _GS * ki
    inb = (j >= 0) & (j < _GS)
    w = jnp.where(inb, lax.bitcast_convert_type((127 - j) << 23, jnp.float32),
                  0.0)
    sel = (xt == m[None, :]).astype(jnp.float32)
    d = lax.dot_general(w, sel, (((1,), (0,)), ((), ())),
                        preferred_element_type=jnp.float32)
    bits = lax.bitcast_convert_type(d, jnp.int32)
    efield = lax.shift_right_logical(bits, 23) & 0xFF
    labk = _GS * lax.broadcasted_iota(jnp.int32, (_NG, _NH), 0) + (127 - efield)
    lab = jnp.min(jnp.where(d > 0.0, labk, _C), axis=0)
    padh = _R // 2 * _L - _NH
    mp = jnp.concatenate([m, jnp.full((padh,), _NEG_INF, jnp.float32)])
    lp = jnp.concatenate([lab, jnp.zeros((padh,), jnp.int32)])
    s_ref[b, pl.ds(h * (_R // 2), _R // 2), :] = mp.reshape(_R // 2, _L)
    l_ref[b, pl.ds(h * (_R // 2), _R // 2), :] = lp.reshape(_R // 2, _L)


def _fused_kernel(cls_ref, b_ref, os_ref, ox1_ref, oy1_ref, ox2_ref,
                  oy2_ref, ol_ref, s_ref, l_ref):
    i = pl.program_id(0)

    @pl.when(i < 2 * _B)
    def _scores_phase():
        _scores_body(cls_ref, s_ref, l_ref, i // 2, i % 2)

    @pl.when(i == 2 * _B)
    def _nms_phase():
        _nms_body(s_ref, b_ref, l_ref, os_ref, ox1_ref, oy1_ref, ox2_ref,
                  oy2_ref, ol_ref)


def _nms_body(s_ref, b_ref, l_ref, os_ref, ox1_ref, oy1_ref, ox2_ref,
              oy2_ref, ol_ref):
    scores = s_ref[...]
    x1 = b_ref[:, 0]
    y1 = b_ref[:, 1]
    x2 = b_ref[:, 2]
    y2 = b_ref[:, 3]
    labs = l_ref[...]

    lin = (lax.broadcasted_iota(jnp.int32, (_B, _R, _L), 1) * _L
           + lax.broadcasted_iota(jnp.int32, (_B, _R, _L), 2))
    lane = lax.broadcasted_iota(jnp.int32, (_B, 1, _L), 2)
    area = jnp.maximum(x2 - x1, 0.0) * jnp.maximum(y2 - y1, 0.0)

    work0 = jnp.where(scores > _SCORE_T, scores, _NEG_INF)
    cm0 = jnp.max(work0, axis=1, keepdims=True)
    zf = jnp.full((_B, 1, _L), -1.0, dtype=jnp.float32)
    zi = jnp.full((_B, 1, _L), -1, dtype=jnp.int32)

    lane1 = lax.broadcasted_iota(jnp.int32, (1, _L), 1)

    def body(k, carry):
        work, cm, o_s, o_x1, o_y1, o_x2, o_y2, o_l = carry
        m = jnp.max(cm, axis=2, keepdims=True)
        tied = work == m
        idx = jnp.min(jnp.where(tied, lin, _NPAD), axis=(1, 2),
                      keepdims=True)
        parts = [[] for _ in range(5)]
        for b in range(_B):
            rb = idx[b, 0, 0] // _L
            lb = idx[b, 0, 0] % _L
            hitl = lane1 == lb
            rows4 = b_ref[b, :, pl.ds(rb, 1), :]
            sel4 = jnp.sum(jnp.where(hitl[None], rows4, 0.0),
                           axis=2, keepdims=True)
            for c in range(4):
                parts[c].append(sel4[c])
            rowl = l_ref[b, pl.ds(rb, 1), :]
            parts[4].append(jnp.sum(jnp.where(hitl, rowl, 0),
                                    axis=1, keepdims=True))
        bx1 = jnp.stack(parts[0], axis=0)
        by1 = jnp.stack(parts[1], axis=0)
        bx2 = jnp.stack(parts[2], axis=0)
        by2 = jnp.stack(parts[3], axis=0)
        blab = jnp.stack(parts[4], axis=0)

        ix1 = jnp.maximum(bx1, x1)
        iy1 = jnp.maximum(by1, y1)
        ix2 = jnp.minimum(bx2, x2)
        iy2 = jnp.minimum(by2, y2)
        inter = jnp.maximum(ix2 - ix1, 0.0) * jnp.maximum(iy2 - iy1, 0.0)
        a1 = jnp.maximum(bx2 - bx1, 0.0) * jnp.maximum(by2 - by1, 0.0)
        iou = inter / (a1 + area - inter + 1e-8)
        sup = iou > _NMS_T
        work = jnp.where(sup, _NEG_INF, work)
        cm = jnp.max(work, axis=1, keepdims=True)

        valid = m > _NEG_INF
        hit = lane == k
        o_s = jnp.where(hit, jnp.where(valid, m, -1.0), o_s)
        o_x1 = jnp.where(hit, jnp.where(valid, bx1, -1.0), o_x1)
        o_y1 = jnp.where(hit, jnp.where(valid, by1, -1.0), o_y1)
        o_x2 = jnp.where(hit, jnp.where(valid, bx2, -1.0), o_x2)
        o_y2 = jnp.where(hit, jnp.where(valid, by2, -1.0), o_y2)
        o_l = jnp.where(hit, jnp.where(valid, blab, -1), o_l)
        return work, cm, o_s, o_x1, o_y1, o_x2, o_y2, o_l

    carry = (work0, cm0, zf, zf, zf, zf, zf, zi)
    _, _, o_s, o_x1, o_y1, o_x2, o_y2, o_l = lax.fori_loop(
        0, _MAXDET, body, carry)
    os_ref[...] = o_s
    ox1_ref[...] = o_x1
    oy1_ref[...] = o_y1
    ox2_ref[...] = o_x2
    oy2_ref[...] = o_y2
    ol_ref[...] = o_l


@jax.jit
def kernel(boxes, classification):
    padh = _R // 2 * _L - _NH
    b_p = jnp.pad(jnp.moveaxis(boxes, 2, 1).reshape(_B, 4, 2, _NH),
                  ((0, 0), (0, 0), (0, 0), (0, padh)))
    b_p = b_p.reshape(_B, 4, _R, _L)

    outs = pl.pallas_call(
        _fused_kernel,
        grid=(2 * _B + 1,),
        in_specs=[
            pl.BlockSpec((1, _NH, _C),
                         lambda i: (jnp.minimum(i, 2 * _B - 1) // 2,
                                    jnp.minimum(i, 2 * _B - 1) % 2, 0)),
            pl.BlockSpec((_B, 4, _R, _L), lambda i: (0, 0, 0, 0)),
        ],
        out_specs=[pl.BlockSpec((_B, 1, _L), lambda i: (0, 0, 0))] * 6,
        out_shape=[jax.ShapeDtypeStruct((_B, 1, _L), jnp.float32)] * 5
        + [jax.ShapeDtypeStruct((_B, 1, _L), jnp.int32)],
        scratch_shapes=[
            pltpu.VMEM((_B, _R, _L), jnp.float32),
            pltpu.VMEM((_B, _R, _L), jnp.int32),
        ],
        interpret=_INTERPRET,
    )(classification, b_p)
    o_s, o_x1, o_y1, o_x2, o_y2, o_l = outs

    out_scores = o_s[:, 0, :_MAXDET]
    out_labels = o_l[:, 0, :_MAXDET]
    out_boxes = jnp.stack(
        [o_x1[:, 0, :_MAXDET], o_y1[:, 0, :_MAXDET],
         o_x2[:, 0, :_MAXDET], o_y2[:, 0, :_MAXDET]], axis=-1)
    return out_boxes, out_scores, out_labels


# speedup vs baseline: 1.0482x; 1.0482x over previous
import functools

import jax
from jax import lax
import jax.numpy as jnp
from jax.experimental import pallas as pl
from jax.experimental.pallas import tpu as pltpu

_B, _N, _C = 8, 20000, 80
_NMS_T = 0.5
_SCORE_T = 0.05
_MAXDET = 100
_R, _L = 160, 128
_NPAD = _R * _L
_NEG_INF = float("-inf")
_NG = 4
_GS = _C // _NG

_INTERPRET = False


def _scores_body(cls_ref, s_ref, l_ref, i):
    x = cls_ref[0]
    xt = x.T
    m = jnp.max(xt, axis=0)

    ci = lax.broadcasted_iota(jnp.int32, (_NG, _C), 1)
    ki = lax.broadcasted_iota(jnp.int32, (_NG, _C), 0)
    j = ci - _GS * ki
    inb = (j >= 0) & (j < _GS)
    w = jnp.where(inb, lax.bitcast_convert_type((127 - j) << 23, jnp.float32),
                  0.0)
    sel = (xt == m[None, :]).astype(jnp.float32)
    d = lax.dot_general(w, sel, (((1,), (0,)), ((), ())),
                        preferred_element_type=jnp.float32)
    bits = lax.bitcast_convert_type(d, jnp.int32)
    efield = lax.shift_right_logical(bits, 23) & 0xFF
    labk = _GS * lax.broadcasted_iota(jnp.int32, (_NG, _N), 0) + (127 - efield)
    lab = jnp.min(jnp.where(d > 0.0, labk, _C), axis=0)
    mp = jnp.concatenate([m, jnp.full((_NPAD - _N,), _NEG_INF, jnp.float32)])
    lp = jnp.concatenate([lab, jnp.zeros((_NPAD - _N,), jnp.int32)])
    s_ref[i] = mp.reshape(_R, _L)
    l_ref[i] = lp.reshape(_R, _L)


def _fused_kernel(cls_ref, b_ref, os_ref, ox1_ref, oy1_ref, ox2_ref,
                  oy2_ref, ol_ref, s_ref, l_ref):
    i = pl.program_id(0)

    @pl.when(i < _B)
    def _scores_phase():
        _scores_body(cls_ref, s_ref, l_ref, i)

    @pl.when(i == _B)
    def _nms_phase():
        _nms_body(s_ref, b_ref, l_ref, os_ref, ox1_ref, oy1_ref, ox2_ref,
                  oy2_ref, ol_ref)


def _nms_body(s_ref, b_ref, l_ref, os_ref, ox1_ref, oy1_ref, ox2_ref,
              oy2_ref, ol_ref):
    scores = s_ref[...]
    x1 = b_ref[:, 0]
    y1 = b_ref[:, 1]
    x2 = b_ref[:, 2]
    y2 = b_ref[:, 3]
    labs = l_ref[...]

    lin = (lax.broadcasted_iota(jnp.int32, (_B, _R, _L), 1) * _L
           + lax.broadcasted_iota(jnp.int32, (_B, _R, _L), 2))
    lane = lax.broadcasted_iota(jnp.int32, (_B, 1, _L), 2)
    area = jnp.maximum(x2 - x1, 0.0) * jnp.maximum(y2 - y1, 0.0)

    work0 = jnp.where(scores > _SCORE_T, scores, _NEG_INF)
    cm0 = jnp.max(work0, axis=1, keepdims=True)
    zf = jnp.full((_B, 1, _L), -1.0, dtype=jnp.float32)
    zi = jnp.full((_B, 1, _L), -1, dtype=jnp.int32)

    lane1 = lax.broadcasted_iota(jnp.int32, (1, _L), 1)

    def body(k, carry):
        work, cm, o_s, o_x1, o_y1, o_x2, o_y2, o_l = carry
        m = jnp.max(cm, axis=2, keepdims=True)
        tied = work == m
        idx = jnp.min(jnp.where(tied, lin, _NPAD), axis=(1, 2),
                      keepdims=True)
        parts = [[] for _ in range(5)]
        for b in range(_B):
            rb = idx[b, 0, 0] // _L
            lb = idx[b, 0, 0] % _L
            hitl = lane1 == lb
            rows4 = b_ref[b, :, pl.ds(rb, 1), :]
            sel4 = jnp.sum(jnp.where(hitl[None], rows4, 0.0),
                           axis=2, keepdims=True)
            for c in range(4):
                parts[c].append(sel4[c])
            rowl = l_ref[b, pl.ds(rb, 1), :]
            parts[4].append(jnp.sum(jnp.where(hitl, rowl, 0),
                                    axis=1, keepdims=True))
        bx1 = jnp.stack(parts[0], axis=0)
        by1 = jnp.stack(parts[1], axis=0)
        bx2 = jnp.stack(parts[2], axis=0)
        by2 = jnp.stack(parts[3], axis=0)
        blab = jnp.stack(parts[4], axis=0)

        ix1 = jnp.maximum(bx1, x1)
        iy1 = jnp.maximum(by1, y1)
        ix2 = jnp.minimum(bx2, x2)
        iy2 = jnp.minimum(by2, y2)
        inter = jnp.maximum(ix2 - ix1, 0.0) * jnp.maximum(iy2 - iy1, 0.0)
        a1 = jnp.maximum(bx2 - bx1, 0.0) * jnp.maximum(by2 - by1, 0.0)
        iou = inter / (a1 + area - inter + 1e-8)
        sup = iou > _NMS_T
        work = jnp.where(sup, _NEG_INF, work)
        cm = jnp.max(work, axis=1, keepdims=True)

        valid = m > _NEG_INF
        hit = lane == k
        o_s = jnp.where(hit, jnp.where(valid, m, -1.0), o_s)
        o_x1 = jnp.where(hit, jnp.where(valid, bx1, -1.0), o_x1)
        o_y1 = jnp.where(hit, jnp.where(valid, by1, -1.0), o_y1)
        o_x2 = jnp.where(hit, jnp.where(valid, bx2, -1.0), o_x2)
        o_y2 = jnp.where(hit, jnp.where(valid, by2, -1.0), o_y2)
        o_l = jnp.where(hit, jnp.where(valid, blab, -1), o_l)
        return work, cm, o_s, o_x1, o_y1, o_x2, o_y2, o_l

    carry = (work0, cm0, zf, zf, zf, zf, zf, zi)
    _, _, o_s, o_x1, o_y1, o_x2, o_y2, o_l = lax.fori_loop(
        0, _MAXDET, body, carry)
    os_ref[...] = o_s
    ox1_ref[...] = o_x1
    oy1_ref[...] = o_y1
    ox2_ref[...] = o_x2
    oy2_ref[...] = o_y2
    ol_ref[...] = o_l


@jax.jit
def kernel(boxes, classification):
    pad = _NPAD - _N
    b_p = jnp.pad(jnp.moveaxis(boxes, 2, 1), ((0, 0), (0, 0), (0, pad)))
    b_p = b_p.reshape(_B, 4, _R, _L)

    outs = pl.pallas_call(
        _fused_kernel,
        grid=(_B + 1,),
        in_specs=[
            pl.BlockSpec((1, _N, _C), lambda i: (min(i, _B - 1)
                                                 if isinstance(i, int)
                                                 else jnp.minimum(i, _B - 1),
                                                 0, 0)),
            pl.BlockSpec((_B, 4, _R, _L), lambda i: (0, 0, 0, 0)),
        ],
        out_specs=[pl.BlockSpec((_B, 1, _L), lambda i: (0, 0, 0))] * 6,
        out_shape=[jax.ShapeDtypeStruct((_B, 1, _L), jnp.float32)] * 5
        + [jax.ShapeDtypeStruct((_B, 1, _L), jnp.int32)],
        scratch_shapes=[
            pltpu.VMEM((_B, _R, _L), jnp.float32),
            pltpu.VMEM((_B, _R, _L), jnp.int32),
        ],
        interpret=_INTERPRET,
    )(classification, b_p)
    o_s, o_x1, o_y1, o_x2, o_y2, o_l = outs

    out_scores = o_s[:, 0, :_MAXDET]
    out_labels = o_l[:, 0, :_MAXDET]
    out_boxes = jnp.stack(
        [o_x1[:, 0, :_MAXDET], o_y1[:, 0, :_MAXDET],
         o_x2[:, 0, :_MAXDET], o_y2[:, 0, :_MAXDET]], axis=-1)
    return out_boxes, out_scores, out_labels


# speedup vs baseline: 1.0839x; 1.0341x over previous
import functools

import jax
from jax import lax
import jax.numpy as jnp
from jax.experimental import pallas as pl
from jax.experimental.pallas import tpu as pltpu

_B, _N, _C = 8, 20000, 80
_NMS_T = 0.5
_SCORE_T = 0.05
_MAXDET = 100
_R, _L = 160, 128
_NPAD = _R * _L
_NEG_INF = float("-inf")
_NG = 4
_GS = _C // _NG

_INTERPRET = False


def _scores_body(cls_ref, s_ref, l_ref, i):
    x = cls_ref[0]
    xt = x.T
    m = jnp.max(xt, axis=0)

    ci = lax.broadcasted_iota(jnp.int32, (_NG, _C), 1)
    ki = lax.broadcasted_iota(jnp.int32, (_NG, _C), 0)
    j = ci - _GS * ki
    inb = (j >= 0) & (j < _GS)
    w = jnp.where(inb, lax.bitcast_convert_type((127 - j) << 23, jnp.float32),
                  0.0)
    sel = (xt == m[None, :]).astype(jnp.float32)
    d = lax.dot_general(w, sel, (((1,), (0,)), ((), ())),
                        preferred_element_type=jnp.float32)
    bits = lax.bitcast_convert_type(d, jnp.int32)
    efield = lax.shift_right_logical(bits, 23) & 0xFF
    labk = _GS * lax.broadcasted_iota(jnp.int32, (_NG, _N), 0) + (127 - efield)
    lab = jnp.min(jnp.where(d > 0.0, labk, _C), axis=0)
    mp = jnp.concatenate([m, jnp.full((_NPAD - _N,), _NEG_INF, jnp.float32)])
    lp = jnp.concatenate([lab, jnp.zeros((_NPAD - _N,), jnp.int32)])
    s_ref[i] = mp.reshape(_R, _L)
    l_ref[i] = lp.reshape(_R, _L)


def _fused_kernel(cls_ref, b_ref, os_ref, ox1_ref, oy1_ref, ox2_ref,
                  oy2_ref, ol_ref, s_ref, l_ref):
    i = pl.program_id(0)

    @pl.when(i < _B)
    def _scores_phase():
        _scores_body(cls_ref, s_ref, l_ref, i)

    @pl.when(i == _B)
    def _nms_phase():
        _nms_body(s_ref, b_ref, l_ref, os_ref, ox1_ref, oy1_ref, ox2_ref,
                  oy2_ref, ol_ref)


def _nms_body(s_ref, b_ref, l_ref, os_ref, ox1_ref, oy1_ref, ox2_ref,
              oy2_ref, ol_ref):
    scores = s_ref[...]
    x1 = b_ref[:, 0]
    y1 = b_ref[:, 1]
    x2 = b_ref[:, 2]
    y2 = b_ref[:, 3]
    labs = l_ref[...]

    lin = (lax.broadcasted_iota(jnp.int32, (_B, _R, _L), 1) * _L
           + lax.broadcasted_iota(jnp.int32, (_B, _R, _L), 2))
    lane = lax.broadcasted_iota(jnp.int32, (_B, 1, _L), 2)
    area = jnp.maximum(x2 - x1, 0.0) * jnp.maximum(y2 - y1, 0.0)

    work0 = jnp.where(scores > _SCORE_T, scores, _NEG_INF)
    cm0 = jnp.max(work0, axis=1, keepdims=True)
    zf = jnp.full((_B, 1, _L), -1.0, dtype=jnp.float32)
    zi = jnp.full((_B, 1, _L), -1, dtype=jnp.int32)

    lane1 = lax.broadcasted_iota(jnp.int32, (1, _L), 1)

    def body(k, carry):
        work, cm, o_s, o_x1, o_y1, o_x2, o_y2, o_l = carry
        m = jnp.max(cm, axis=2, keepdims=True)
        tied = work == m
        idx = jnp.min(jnp.where(tied, lin, _NPAD), axis=(1, 2),
                      keepdims=True)
        parts = [[] for _ in range(5)]
        for b in range(_B):
            rb = idx[b, 0, 0] // _L
            lb = idx[b, 0, 0] % _L
            hitl = lane1 == lb
            rows4 = b_ref[b, :, pl.ds(rb, 1), :]
            sel4 = jnp.sum(jnp.where(hitl[None], rows4, 0.0),
                           axis=2, keepdims=True)
            for c in range(4):
                parts[c].append(sel4[c])
            rowl = l_ref[b, pl.ds(rb, 1), :]
            parts[4].append(jnp.sum(jnp.where(hitl, rowl, 0),
                                    axis=1, keepdims=True))
        bx1 = jnp.stack(parts[0], axis=0)
        by1 = jnp.stack(parts[1], axis=0)
        bx2 = jnp.stack(parts[2], axis=0)
        by2 = jnp.stack(parts[3], axis=0)
        blab = jnp.stack(parts[4], axis=0)

        ix1 = jnp.maximum(bx1, x1)
        iy1 = jnp.maximum(by1, y1)
        ix2 = jnp.minimum(bx2, x2)
        iy2 = jnp.minimum(by2, y2)
        inter = jnp.maximum(ix2 - ix1, 0.0) * jnp.maximum(iy2 - iy1, 0.0)
        a1 = jnp.maximum(bx2 - bx1, 0.0) * jnp.maximum(by2 - by1, 0.0)
        iou = inter / (a1 + area - inter + 1e-8)
        sup = iou > _NMS_T
        work = jnp.where(sup, _NEG_INF, work)
        cm = jnp.max(work, axis=1, keepdims=True)

        valid = m > _NEG_INF
        hit = lane == k
        o_s = jnp.where(hit, jnp.where(valid, m, -1.0), o_s)
        o_x1 = jnp.where(hit, jnp.where(valid, bx1, -1.0), o_x1)
        o_y1 = jnp.where(hit, jnp.where(valid, by1, -1.0), o_y1)
        o_x2 = jnp.where(hit, jnp.where(valid, bx2, -1.0), o_x2)
        o_y2 = jnp.where(hit, jnp.where(valid, by2, -1.0), o_y2)
        o_l = jnp.where(hit, jnp.where(valid, blab, -1), o_l)
        return work, cm, o_s, o_x1, o_y1, o_x2, o_y2, o_l

    carry = (work0, cm0, zf, zf, zf, zf, zf, zi)
    _, _, o_s, o_x1, o_y1, o_x2, o_y2, o_l = lax.fori_loop(
        0, _MAXDET, body, carry, unroll=2)
    os_ref[...] = o_s
    ox1_ref[...] = o_x1
    oy1_ref[...] = o_y1
    ox2_ref[...] = o_x2
    oy2_ref[...] = o_y2
    ol_ref[...] = o_l


@jax.jit
def kernel(boxes, classification):
    pad = _NPAD - _N
    b_p = jnp.pad(jnp.moveaxis(boxes, 2, 1), ((0, 0), (0, 0), (0, pad)))
    b_p = b_p.reshape(_B, 4, _R, _L)

    outs = pl.pallas_call(
        _fused_kernel,
        grid=(_B + 1,),
        in_specs=[
            pl.BlockSpec((1, _N, _C), lambda i: (min(i, _B - 1)
                                                 if isinstance(i, int)
                                                 else jnp.minimum(i, _B - 1),
                                                 0, 0)),
            pl.BlockSpec((_B, 4, _R, _L), lambda i: (0, 0, 0, 0)),
        ],
        out_specs=[pl.BlockSpec((_B, 1, _L), lambda i: (0, 0, 0))] * 6,
        out_shape=[jax.ShapeDtypeStruct((_B, 1, _L), jnp.float32)] * 5
        + [jax.ShapeDtypeStruct((_B, 1, _L), jnp.int32)],
        scratch_shapes=[
            pltpu.VMEM((_B, _R, _L), jnp.float32),
            pltpu.VMEM((_B, _R, _L), jnp.int32),
        ],
        interpret=_INTERPRET,
    )(classification, b_p)
    o_s, o_x1, o_y1, o_x2, o_y2, o_l = outs

    out_scores = o_s[:, 0, :_MAXDET]
    out_labels = o_l[:, 0, :_MAXDET]
    out_boxes = jnp.stack(
        [o_x1[:, 0, :_MAXDET], o_y1[:, 0, :_MAXDET],
         o_x2[:, 0, :_MAXDET], o_y2[:, 0, :_MAXDET]], axis=-1)
    return out_boxes, out_scores, out_labels


# speedup vs baseline: 1.1037x; 1.0183x over previous
import functools

import jax
from jax import lax
import jax.numpy as jnp
from jax.experimental import pallas as pl
from jax.experimental.pallas import tpu as pltpu

_B, _N, _C = 8, 20000, 80
_NMS_T = 0.5
_SCORE_T = 0.05
_MAXDET = 100
_R, _L = 160, 128
_NPAD = _R * _L
_NEG_INF = float("-inf")
_NG = 4
_GS = _C // _NG

_INTERPRET = False


def _scores_body(cls_ref, s_ref, l_ref, i):
    x = cls_ref[0]
    xt = x.T
    m = jnp.max(xt, axis=0)

    ci = lax.broadcasted_iota(jnp.int32, (_NG, _C), 1)
    ki = lax.broadcasted_iota(jnp.int32, (_NG, _C), 0)
    j = ci - _GS * ki
    inb = (j >= 0) & (j < _GS)
    w = jnp.where(inb, lax.bitcast_convert_type((127 - j) << 23, jnp.float32),
                  0.0)
    sel = (xt == m[None, :]).astype(jnp.float32)
    d = lax.dot_general(w, sel, (((1,), (0,)), ((), ())),
                        preferred_element_type=jnp.float32)
    bits = lax.bitcast_convert_type(d, jnp.int32)
    efield = lax.shift_right_logical(bits, 23) & 0xFF
    labk = _GS * lax.broadcasted_iota(jnp.int32, (_NG, _N), 0) + (127 - efield)
    lab = jnp.min(jnp.where(d > 0.0, labk, _C), axis=0)
    mp = jnp.concatenate([m, jnp.full((_NPAD - _N,), _NEG_INF, jnp.float32)])
    lp = jnp.concatenate([lab, jnp.zeros((_NPAD - _N,), jnp.int32)])
    s_ref[i] = mp.reshape(_R, _L)
    l_ref[i] = lp.reshape(_R, _L)


def _fused_kernel(cls_ref, b_ref, os_ref, ox1_ref, oy1_ref, ox2_ref,
                  oy2_ref, ol_ref, s_ref, l_ref):
    i = pl.program_id(0)

    @pl.when(i < _B)
    def _scores_phase():
        _scores_body(cls_ref, s_ref, l_ref, i)

    @pl.when(i == _B)
    def _nms_phase():
        _nms_body(s_ref, b_ref, l_ref, os_ref, ox1_ref, oy1_ref, ox2_ref,
                  oy2_ref, ol_ref)


def _nms_body(s_ref, b_ref, l_ref, os_ref, ox1_ref, oy1_ref, ox2_ref,
              oy2_ref, ol_ref):
    scores = s_ref[...]
    x1 = b_ref[:, 0]
    y1 = b_ref[:, 1]
    x2 = b_ref[:, 2]
    y2 = b_ref[:, 3]
    labs = l_ref[...]

    lin = (lax.broadcasted_iota(jnp.int32, (_B, _R, _L), 1) * _L
           + lax.broadcasted_iota(jnp.int32, (_B, _R, _L), 2))
    lane = lax.broadcasted_iota(jnp.int32, (_B, 1, _L), 2)
    area = jnp.maximum(x2 - x1, 0.0) * jnp.maximum(y2 - y1, 0.0)

    work0 = jnp.where(scores > _SCORE_T, scores, _NEG_INF)
    cm0 = jnp.max(work0, axis=1, keepdims=True)
    zf = jnp.full((_B, 1, _L), -1.0, dtype=jnp.float32)
    zi = jnp.full((_B, 1, _L), -1, dtype=jnp.int32)

    lane1 = lax.broadcasted_iota(jnp.int32, (1, _L), 1)

    def body(k, carry):
        work, cm, o_s, o_x1, o_y1, o_x2, o_y2, o_l = carry
        m = jnp.max(cm, axis=2, keepdims=True)
        tied = work == m
        idx = jnp.min(jnp.where(tied, lin, _NPAD), axis=(1, 2),
                      keepdims=True)
        parts = [[] for _ in range(5)]
        for b in range(_B):
            rb = idx[b, 0, 0] // _L
            lb = idx[b, 0, 0] % _L
            hitl = lane1 == lb
            rows4 = b_ref[b, :, pl.ds(rb, 1), :]
            sel4 = jnp.sum(jnp.where(hitl[None], rows4, 0.0),
                           axis=2, keepdims=True)
            for c in range(4):
                parts[c].append(sel4[c])
            rowl = l_ref[b, pl.ds(rb, 1), :]
            parts[4].append(jnp.sum(jnp.where(hitl, rowl, 0),
                                    axis=1, keepdims=True))
        bx1 = jnp.stack(parts[0], axis=0)
        by1 = jnp.stack(parts[1], axis=0)
        bx2 = jnp.stack(parts[2], axis=0)
        by2 = jnp.stack(parts[3], axis=0)
        blab = jnp.stack(parts[4], axis=0)

        ix1 = jnp.maximum(bx1, x1)
        iy1 = jnp.maximum(by1, y1)
        ix2 = jnp.minimum(bx2, x2)
        iy2 = jnp.minimum(by2, y2)
        inter = jnp.maximum(ix2 - ix1, 0.0) * jnp.maximum(iy2 - iy1, 0.0)
        a1 = jnp.maximum(bx2 - bx1, 0.0) * jnp.maximum(by2 - by1, 0.0)
        iou = inter / (a1 + area - inter + 1e-8)
        sup = iou > _NMS_T
        work = jnp.where(sup, _NEG_INF, work)
        cm = jnp.max(work, axis=1, keepdims=True)

        valid = m > _NEG_INF
        hit = lane == k
        o_s = jnp.where(hit, jnp.where(valid, m, -1.0), o_s)
        o_x1 = jnp.where(hit, jnp.where(valid, bx1, -1.0), o_x1)
        o_y1 = jnp.where(hit, jnp.where(valid, by1, -1.0), o_y1)
        o_x2 = jnp.where(hit, jnp.where(valid, bx2, -1.0), o_x2)
        o_y2 = jnp.where(hit, jnp.where(valid, by2, -1.0), o_y2)
        o_l = jnp.where(hit, jnp.where(valid, blab, -1), o_l)
        return work, cm, o_s, o_x1, o_y1, o_x2, o_y2, o_l

    carry = (work0, cm0, zf, zf, zf, zf, zf, zi)
    _, _, o_s, o_x1, o_y1, o_x2, o_y2, o_l = lax.fori_loop(
        0, _MAXDET, body, carry, unroll=4)
    os_ref[...] = o_s
    ox1_ref[...] = o_x1
    oy1_ref[...] = o_y1
    ox2_ref[...] = o_x2
    oy2_ref[...] = o_y2
    ol_ref[...] = o_l


@jax.jit
def kernel(boxes, classification):
    pad = _NPAD - _N
    b_p = jnp.pad(jnp.moveaxis(boxes, 2, 1), ((0, 0), (0, 0), (0, pad)))
    b_p = b_p.reshape(_B, 4, _R, _L)

    outs = pl.pallas_call(
        _fused_kernel,
        grid=(_B + 1,),
        in_specs=[
            pl.BlockSpec((1, _N, _C), lambda i: (min(i, _B - 1)
                                                 if isinstance(i, int)
                                                 else jnp.minimum(i, _B - 1),
                                                 0, 0)),
            pl.BlockSpec((_B, 4, _R, _L), lambda i: (0, 0, 0, 0)),
        ],
        out_specs=[pl.BlockSpec((_B, 1, _L), lambda i: (0, 0, 0))] * 6,
        out_shape=[jax.ShapeDtypeStruct((_B, 1, _L), jnp.float32)] * 5
        + [jax.ShapeDtypeStruct((_B, 1, _L), jnp.int32)],
        scratch_shapes=[
            pltpu.VMEM((_B, _R, _L), jnp.float32),
            pltpu.VMEM((_B, _R, _L), jnp.int32),
        ],
        interpret=_INTERPRET,
    )(classification, b_p)
    o_s, o_x1, o_y1, o_x2, o_y2, o_l = outs

    out_scores = o_s[:, 0, :_MAXDET]
    out_labels = o_l[:, 0, :_MAXDET]
    out_boxes = jnp.stack(
        [o_x1[:, 0, :_MAXDET], o_y1[:, 0, :_MAXDET],
         o_x2[:, 0, :_MAXDET], o_y2[:, 0, :_MAXDET]], axis=-1)
    return out_boxes, out_scores, out_labels


# speedup vs baseline: 1.1091x; 1.0049x over previous
import functools

import jax
from jax import lax
import jax.numpy as jnp
from jax.experimental import pallas as pl
from jax.experimental.pallas import tpu as pltpu

_B, _N, _C = 8, 20000, 80
_NMS_T = 0.5
_SCORE_T = 0.05
_MAXDET = 100
_R, _L = 160, 128
_NPAD = _R * _L
_NEG_INF = float("-inf")
_NG = 4
_GS = _C // _NG

_INTERPRET = False


def _scores_body(cls_ref, s_ref, l_ref, i):
    x = cls_ref[0]
    xt = x.T
    m = jnp.max(xt, axis=0)

    ci = lax.broadcasted_iota(jnp.int32, (_NG, _C), 1)
    ki = lax.broadcasted_iota(jnp.int32, (_NG, _C), 0)
    j = ci - _GS * ki
    inb = (j >= 0) & (j < _GS)
    w = jnp.where(inb, lax.bitcast_convert_type((127 - j) << 23, jnp.float32),
                  0.0)
    sel = (xt == m[None, :]).astype(jnp.float32)
    d = lax.dot_general(w, sel, (((1,), (0,)), ((), ())),
                        preferred_element_type=jnp.float32)
    bits = lax.bitcast_convert_type(d, jnp.int32)
    efield = lax.shift_right_logical(bits, 23) & 0xFF
    labk = _GS * lax.broadcasted_iota(jnp.int32, (_NG, _N), 0) + (127 - efield)
    lab = jnp.min(jnp.where(d > 0.0, labk, _C), axis=0)
    mp = jnp.concatenate([m, jnp.full((_NPAD - _N,), _NEG_INF, jnp.float32)])
    lp = jnp.concatenate([lab, jnp.zeros((_NPAD - _N,), jnp.int32)])
    s_ref[i] = mp.reshape(_R, _L)
    l_ref[i] = lp.reshape(_R, _L)


def _fused_kernel(cls_ref, b_ref, os_ref, ox1_ref, oy1_ref, ox2_ref,
                  oy2_ref, ol_ref, s_ref, l_ref):
    i = pl.program_id(0)

    @pl.when(i < _B)
    def _scores_phase():
        _scores_body(cls_ref, s_ref, l_ref, i)

    @pl.when(i == _B)
    def _nms_phase():
        _nms_body(s_ref, b_ref, l_ref, os_ref, ox1_ref, oy1_ref, ox2_ref,
                  oy2_ref, ol_ref)


def _nms_body(s_ref, b_ref, l_ref, os_ref, ox1_ref, oy1_ref, ox2_ref,
              oy2_ref, ol_ref):
    scores = s_ref[...]
    x1 = b_ref[:, 0]
    y1 = b_ref[:, 1]
    x2 = b_ref[:, 2]
    y2 = b_ref[:, 3]
    labs = l_ref[...]

    lin = (lax.broadcasted_iota(jnp.int32, (_B, _R, _L), 1) * _L
           + lax.broadcasted_iota(jnp.int32, (_B, _R, _L), 2))
    lane = lax.broadcasted_iota(jnp.int32, (_B, 1, _L), 2)
    area = jnp.maximum(x2 - x1, 0.0) * jnp.maximum(y2 - y1, 0.0)

    work0 = jnp.where(scores > _SCORE_T, scores, _NEG_INF)
    cm0 = jnp.max(work0, axis=1, keepdims=True)
    zf = jnp.full((_B, 1, _L), -1.0, dtype=jnp.float32)
    zi = jnp.full((_B, 1, _L), -1, dtype=jnp.int32)

    lane1 = lax.broadcasted_iota(jnp.int32, (1, _L), 1)

    def body(k, carry):
        work, cm, o_s, o_x1, o_y1, o_x2, o_y2, o_l = carry
        m = jnp.max(cm, axis=2, keepdims=True)
        tied = work == m
        idx = jnp.min(jnp.where(tied, lin, _NPAD), axis=(1, 2),
                      keepdims=True)
        parts = [[] for _ in range(5)]
        for b in range(_B):
            rb = idx[b, 0, 0] // _L
            lb = idx[b, 0, 0] % _L
            hitl = lane1 == lb
            rows4 = b_ref[b, :, pl.ds(rb, 1), :]
            sel4 = jnp.sum(jnp.where(hitl[None], rows4, 0.0),
                           axis=2, keepdims=True)
            for c in range(4):
                parts[c].append(sel4[c])
            rowl = l_ref[b, pl.ds(rb, 1), :]
            parts[4].append(jnp.sum(jnp.where(hitl, rowl, 0),
                                    axis=1, keepdims=True))
        bx1 = jnp.stack(parts[0], axis=0)
        by1 = jnp.stack(parts[1], axis=0)
        bx2 = jnp.stack(parts[2], axis=0)
        by2 = jnp.stack(parts[3], axis=0)
        blab = jnp.stack(parts[4], axis=0)

        ix1 = jnp.maximum(bx1, x1)
        iy1 = jnp.maximum(by1, y1)
        ix2 = jnp.minimum(bx2, x2)
        iy2 = jnp.minimum(by2, y2)
        inter = jnp.maximum(ix2 - ix1, 0.0) * jnp.maximum(iy2 - iy1, 0.0)
        a1 = jnp.maximum(bx2 - bx1, 0.0) * jnp.maximum(by2 - by1, 0.0)
        iou = inter / (a1 + area - inter + 1e-8)
        sup = iou > _NMS_T
        work = jnp.where(sup, _NEG_INF, work)
        cm = jnp.max(work, axis=1, keepdims=True)

        valid = m > _NEG_INF
        hit = lane == k
        o_s = jnp.where(hit, jnp.where(valid, m, -1.0), o_s)
        o_x1 = jnp.where(hit, jnp.where(valid, bx1, -1.0), o_x1)
        o_y1 = jnp.where(hit, jnp.where(valid, by1, -1.0), o_y1)
        o_x2 = jnp.where(hit, jnp.where(valid, bx2, -1.0), o_x2)
        o_y2 = jnp.where(hit, jnp.where(valid, by2, -1.0), o_y2)
        o_l = jnp.where(hit, jnp.where(valid, blab, -1), o_l)
        return work, cm, o_s, o_x1, o_y1, o_x2, o_y2, o_l

    carry = (work0, cm0, zf, zf, zf, zf, zf, zi)
    _, _, o_s, o_x1, o_y1, o_x2, o_y2, o_l = lax.fori_loop(
        0, _MAXDET, body, carry, unroll=10)
    os_ref[...] = o_s
    ox1_ref[...] = o_x1
    oy1_ref[...] = o_y1
    ox2_ref[...] = o_x2
    oy2_ref[...] = o_y2
    ol_ref[...] = o_l


@jax.jit
def kernel(boxes, classification):
    pad = _NPAD - _N
    b_p = jnp.pad(jnp.moveaxis(boxes, 2, 1), ((0, 0), (0, 0), (0, pad)))
    b_p = b_p.reshape(_B, 4, _R, _L)

    outs = pl.pallas_call(
        _fused_kernel,
        grid=(_B + 1,),
        in_specs=[
            pl.BlockSpec((1, _N, _C), lambda i: (min(i, _B - 1)
                                                 if isinstance(i, int)
                                                 else jnp.minimum(i, _B - 1),
                                                 0, 0)),
            pl.BlockSpec((_B, 4, _R, _L), lambda i: (0, 0, 0, 0)),
        ],
        out_specs=[pl.BlockSpec((_B, 1, _L), lambda i: (0, 0, 0))] * 6,
        out_shape=[jax.ShapeDtypeStruct((_B, 1, _L), jnp.float32)] * 5
        + [jax.ShapeDtypeStruct((_B, 1, _L), jnp.int32)],
        scratch_shapes=[
            pltpu.VMEM((_B, _R, _L), jnp.float32),
            pltpu.VMEM((_B, _R, _L), jnp.int32),
        ],
        interpret=_INTERPRET,
    )(classification, b_p)
    o_s, o_x1, o_y1, o_x2, o_y2, o_l = outs

    out_scores = o_s[:, 0, :_MAXDET]
    out_labels = o_l[:, 0, :_MAXDET]
    out_boxes = jnp.stack(
        [o_x1[:, 0, :_MAXDET], o_y1[:, 0, :_MAXDET],
         o_x2[:, 0, :_MAXDET], o_y2[:, 0, :_MAXDET]], axis=-1)
    return out_boxes, out_scores, out_labels
